# Initial kernel scaffold; baseline (speedup 1.0000x reference)
import jax
import jax.numpy as jnp
from jax.experimental import pallas as pl


def kernel(input_ids, embed_table):
    raise NotImplementedError("write your pallas kernel here")


# speedup vs baseline: 1.3993x; 1.3993x over previous
import functools

import jax
import jax.numpy as jnp
from jax import lax
from jax.experimental import pallas as pl
from jax.experimental.pallas import tpu as pltpu
from jax.experimental.pallas import tpu_sc as plsc

VOCAB = 100000
D_MODEL = 1024
N_IDS = 4 * 2048

_info = plsc.get_sparse_core_info()
_NC, _NS = _info.num_cores, _info.num_subcores
_NW = _NC * _NS
_B_PER_W = N_IDS // _NW
_CHUNK = 32
_N_CHUNKS = _B_PER_W // _CHUNK


def _embed_body(ids_hbm, table_hbm, out_hbm, idx_v, rows_v, sem):
    wid = lax.axis_index("s") * _NC + lax.axis_index("c")
    base = wid * _B_PER_W
    pltpu.sync_copy(ids_hbm.at[pl.ds(base, _B_PER_W)], idx_v)
    for c in range(_N_CHUNKS):
        pltpu.async_copy(
            table_hbm.at[idx_v.at[pl.ds(c * _CHUNK, _CHUNK)]], rows_v, sem
        ).wait()
        pltpu.sync_copy(rows_v, out_hbm.at[pl.ds(base + c * _CHUNK, _CHUNK)])


@functools.partial(
    pl.kernel,
    out_type=jax.ShapeDtypeStruct((N_IDS, D_MODEL), jnp.float32),
    mesh=plsc.VectorSubcoreMesh(core_axis_name="c", subcore_axis_name="s"),
    scratch_types=[
        pltpu.VMEM((_B_PER_W,), jnp.int32),
        pltpu.VMEM((_CHUNK, D_MODEL), jnp.float32),
        pltpu.SemaphoreType.DMA,
    ],
)
def _embed_lookup(ids_hbm, table_hbm, out_hbm, idx_v, rows_v, sem):
    _embed_body(ids_hbm, table_hbm, out_hbm, idx_v, rows_v, sem)


def kernel(input_ids, embed_table):
    batch, seq = input_ids.shape
    flat_ids = input_ids.reshape(-1).astype(jnp.int32)
    out = _embed_lookup(flat_ids, embed_table)
    return out.reshape(batch, seq, D_MODEL)


# speedup vs baseline: 1.5668x; 1.1197x over previous
import functools

import jax
import jax.numpy as jnp
from jax import lax
from jax.experimental import pallas as pl
from jax.experimental.pallas import tpu as pltpu
from jax.experimental.pallas import tpu_sc as plsc

VOCAB = 100000
D_MODEL = 1024
N_IDS = 4 * 2048

_info = plsc.get_sparse_core_info()
_NC, _NS = _info.num_cores, _info.num_subcores
_NW = _NC * _NS
_B_PER_W = N_IDS // _NW
_CHUNK = 32
_N_CHUNKS = _B_PER_W // _CHUNK
_NBUF = 3


def _embed_body(ids_hbm, table_hbm, out_hbm, idx_v, rows_v, sem_g, sem_w):
    wid = lax.axis_index("s") * _NC + lax.axis_index("c")
    base = wid * _B_PER_W
    pltpu.sync_copy(ids_hbm.at[pl.ds(base, _B_PER_W)], idx_v)

    def start_gather(chunk, buf):
        pltpu.async_copy(
            table_hbm.at[idx_v.at[pl.ds(chunk * _CHUNK, _CHUNK)]],
            rows_v.at[buf],
            sem_g.at[buf],
        )

    def write_copy(chunk, buf):
        return pltpu.async_copy(
            rows_v.at[buf],
            out_hbm.at[pl.ds(base + chunk * _CHUNK, _CHUNK)],
            sem_w.at[buf],
        )

    for b in range(_NBUF):
        start_gather(b, b)
    writes = [None] * _NBUF
    for c in range(_N_CHUNKS):
        b = c % _NBUF
        pltpu.make_async_copy(
            table_hbm.at[idx_v.at[pl.ds(c * _CHUNK, _CHUNK)]],
            rows_v.at[b],
            sem_g.at[b],
        ).wait()
        writes[b] = write_copy(c, b)
        nxt = c + _NBUF
        if nxt < _N_CHUNKS:
            writes[b].wait()
            start_gather(nxt, b)
    for c in range(_N_CHUNKS - _NBUF, _N_CHUNKS):
        if c >= 0:
            writes[c % _NBUF].wait()


@functools.partial(
    pl.kernel,
    out_type=jax.ShapeDtypeStruct((N_IDS, D_MODEL), jnp.float32),
    mesh=plsc.VectorSubcoreMesh(core_axis_name="c", subcore_axis_name="s"),
    scratch_types=[
        pltpu.VMEM((_B_PER_W,), jnp.int32),
        pltpu.VMEM((_NBUF, _CHUNK, D_MODEL), jnp.float32),
        pltpu.SemaphoreType.DMA((_NBUF,)),
        pltpu.SemaphoreType.DMA((_NBUF,)),
    ],
)
def _embed_lookup(ids_hbm, table_hbm, out_hbm, idx_v, rows_v, sem_g, sem_w):
    _embed_body(ids_hbm, table_hbm, out_hbm, idx_v, rows_v, sem_g, sem_w)


def kernel(input_ids, embed_table):
    batch, seq = input_ids.shape
    flat_ids = input_ids.reshape(-1).astype(jnp.int32)
    out = _embed_lookup(flat_ids, embed_table)
    return out.reshape(batch, seq, D_MODEL)


# speedup vs baseline: 1.5839x; 1.0109x over previous
import functools

import jax
import jax.numpy as jnp
from jax import lax
from jax.experimental import pallas as pl
from jax.experimental.pallas import tpu as pltpu
from jax.experimental.pallas import tpu_sc as plsc

VOCAB = 100000
D_MODEL = 1024
N_IDS = 4 * 2048

_info = plsc.get_sparse_core_info()
_NC, _NS = _info.num_cores, _info.num_subcores
_NW = _NC * _NS
_B_PER_W = N_IDS // _NW
_CHUNK = 16
_N_CHUNKS = _B_PER_W // _CHUNK
_NBUF = 6


def _embed_body(ids_hbm, table_hbm, out_hbm, idx_v, rows_v, sem_g, sem_w):
    wid = lax.axis_index("s") * _NC + lax.axis_index("c")
    base = wid * _B_PER_W
    pltpu.sync_copy(ids_hbm.at[pl.ds(base, _B_PER_W)], idx_v)

    def start_gather(chunk, buf):
        pltpu.async_copy(
            table_hbm.at[idx_v.at[pl.ds(chunk * _CHUNK, _CHUNK)]],
            rows_v.at[buf],
            sem_g.at[buf],
        )

    def write_copy(chunk, buf):
        return pltpu.async_copy(
            rows_v.at[buf],
            out_hbm.at[pl.ds(base + chunk * _CHUNK, _CHUNK)],
            sem_w.at[buf],
        )

    for b in range(_NBUF):
        start_gather(b, b)
    writes = [None] * _NBUF
    for c in range(_N_CHUNKS):
        b = c % _NBUF
        pltpu.make_async_copy(
            table_hbm.at[idx_v.at[pl.ds(c * _CHUNK, _CHUNK)]],
            rows_v.at[b],
            sem_g.at[b],
        ).wait()
        writes[b] = write_copy(c, b)
        nxt = c + _NBUF
        if nxt < _N_CHUNKS:
            writes[b].wait()
            start_gather(nxt, b)
    for c in range(_N_CHUNKS - _NBUF, _N_CHUNKS):
        if c >= 0:
            writes[c % _NBUF].wait()


@functools.partial(
    pl.kernel,
    out_type=jax.ShapeDtypeStruct((N_IDS, D_MODEL), jnp.float32),
    mesh=plsc.VectorSubcoreMesh(core_axis_name="c", subcore_axis_name="s"),
    scratch_types=[
        pltpu.VMEM((_B_PER_W,), jnp.int32),
        pltpu.VMEM((_NBUF, _CHUNK, D_MODEL), jnp.float32),
        pltpu.SemaphoreType.DMA((_NBUF,)),
        pltpu.SemaphoreType.DMA((_NBUF,)),
    ],
)
def _embed_lookup(ids_hbm, table_hbm, out_hbm, idx_v, rows_v, sem_g, sem_w):
    _embed_body(ids_hbm, table_hbm, out_hbm, idx_v, rows_v, sem_g, sem_w)


def kernel(input_ids, embed_table):
    batch, seq = input_ids.shape
    flat_ids = input_ids.reshape(-1).astype(jnp.int32)
    out = _embed_lookup(flat_ids, embed_table)
    return out.reshape(batch, seq, D_MODEL)
